# Initial kernel scaffold; baseline (speedup 1.0000x reference)
import jax
import jax.numpy as jnp
from jax.experimental import pallas as pl


def kernel(x, edge_label_index, W1, b1, W2, b2):
    raise NotImplementedError("write your pallas kernel here")


# speedup vs baseline: 22.1236x; 22.1236x over previous
import functools

import jax
import jax.numpy as jnp
from jax import lax
from jax.experimental import pallas as pl
from jax.experimental.pallas import tpu as pltpu
from jax.experimental.pallas import tpu_sc as plsc

DIM = 16
NW = 32
CHUNK = 2000


def _tc_precompute(x, W1, b1row):
    n_nodes = x.shape[0]
    blk = 5000
    grid = (n_nodes // blk,)

    def body(x_ref, w_ref, b_ref, u_ref, v_ref):
        xb = x_ref[...]
        w = w_ref[...]
        u_ref[...] = (
            jnp.dot(xb, w[:DIM, :], preferred_element_type=jnp.float32)
            + b_ref[...]
        )
        v_ref[...] = jnp.dot(xb, w[DIM:, :], preferred_element_type=jnp.float32)

    return pl.pallas_call(
        body,
        grid=grid,
        in_specs=[
            pl.BlockSpec((blk, DIM), lambda i: (i, 0)),
            pl.BlockSpec((2 * DIM, DIM), lambda i: (0, 0)),
            pl.BlockSpec((1, DIM), lambda i: (0, 0)),
        ],
        out_specs=[
            pl.BlockSpec((blk, DIM), lambda i: (i, 0)),
            pl.BlockSpec((blk, DIM), lambda i: (i, 0)),
        ],
        out_shape=[
            jax.ShapeDtypeStruct((n_nodes, DIM), jnp.float32),
            jax.ShapeDtypeStruct((n_nodes, DIM), jnp.float32),
        ],
    )(x, W1, b1row)


def _make_sc_kernel(n_edges):
    per_w = n_edges // NW
    chunk = CHUNK
    n_chunks = per_w // chunk
    groups = chunk // DIM
    info = plsc.get_sparse_core_info()
    nc = info.num_cores
    mesh = plsc.VectorSubcoreMesh(core_axis_name="c", subcore_axis_name="s")

    @functools.partial(
        pl.kernel,
        mesh=mesh,
        out_type=jax.ShapeDtypeStruct((n_edges,), jnp.float32),
        scratch_types=[
            pltpu.VMEM((chunk,), jnp.int32),
            pltpu.VMEM((chunk,), jnp.int32),
            pltpu.VMEM((chunk, DIM), jnp.float32),
            pltpu.VMEM((chunk, DIM), jnp.float32),
            pltpu.VMEM((chunk,), jnp.float32),
            pltpu.VMEM((DIM, DIM), jnp.float32),
            pltpu.VMEM((DIM,), jnp.float32),
            pltpu.SemaphoreType.DMA,
            pltpu.SemaphoreType.DMA,
        ],
        compiler_params=pltpu.CompilerParams(
            needs_layout_passes=False, use_tc_tiling_on_sc=False
        ),
    )
    def sc_kernel(u_hbm, v_hbm, eh_hbm, et_hbm, w2s_hbm, b2s_hbm, out_hbm,
                  idxh_v, idxt_v, ubuf, vbuf, outbuf, w2v, b2v, sem_u, sem_v):
        wid = lax.axis_index("s") * nc + lax.axis_index("c")
        base0 = wid * per_w
        pltpu.sync_copy(w2s_hbm, w2v)
        pltpu.sync_copy(b2s_hbm, b2v)
        w2rows = [w2v[k] for k in range(DIM)]
        b2vec = b2v[...]
        iota16 = lax.iota(jnp.int32, DIM)

        def chunk_body(ci, carry):
            base = base0 + ci * chunk
            pltpu.sync_copy(eh_hbm.at[pl.ds(base, chunk)], idxh_v)
            pltpu.sync_copy(et_hbm.at[pl.ds(base, chunk)], idxt_v)
            cp_u = pltpu.async_copy(u_hbm.at[idxh_v], ubuf, sem_u)
            cp_v = pltpu.async_copy(v_hbm.at[idxt_v], vbuf, sem_v)
            cp_u.wait()
            cp_v.wait()

            def grp_body(g, c2):
                rows = g * DIM + iota16
                acc = b2vec
                for k in range(DIM):
                    colk = jnp.full((DIM,), k, jnp.int32)
                    uu = plsc.load_gather(ubuf, [rows, colk])
                    vv = plsc.load_gather(vbuf, [rows, colk])
                    acc = acc + jnp.maximum(uu + vv, 0.0) * w2rows[k]
                outbuf[pl.ds(g * DIM, DIM)] = 1.0 / (1.0 + jnp.exp(-acc))
                return c2

            lax.fori_loop(0, groups, grp_body, 0)
            pltpu.sync_copy(outbuf, out_hbm.at[pl.ds(base, chunk)])
            return carry

        lax.fori_loop(0, n_chunks, chunk_body, 0)

    return sc_kernel


def kernel(x, edge_label_index, W1, b1, W2, b2):
    n_edges = edge_label_index.shape[1]
    eh = edge_label_index[0].astype(jnp.int32)
    et = edge_label_index[1].astype(jnp.int32)
    U, V = _tc_precompute(x, W1, b1.reshape(1, DIM))
    w2s = jnp.broadcast_to(W2.reshape(DIM, 1), (DIM, DIM))
    b2s = jnp.broadcast_to(b2.reshape(1), (DIM,))
    out = _make_sc_kernel(n_edges)(U, V, eh, et, w2s, b2s)
    return out.reshape(n_edges, 1)


# speedup vs baseline: 29.6624x; 1.3408x over previous
import functools

import jax
import jax.numpy as jnp
from jax import lax
from jax.experimental import pallas as pl
from jax.experimental.pallas import tpu as pltpu
from jax.experimental.pallas import tpu_sc as plsc

DIM = 16
PAD = 17
NW = 32
CHUNK = 2000


def _tc_precompute(x, W1, b1row):
    n_nodes = x.shape[0]
    blk = 5000
    grid = (n_nodes // blk,)

    def body(x_ref, w_ref, b_ref, u_ref, v_ref):
        xb = x_ref[...]
        w = w_ref[...]
        u_ref[:, 0:DIM] = (
            jnp.dot(xb, w[:DIM, :], preferred_element_type=jnp.float32)
            + b_ref[...]
        )
        u_ref[:, DIM:PAD] = jnp.zeros((blk, PAD - DIM), jnp.float32)
        v_ref[:, 0:DIM] = jnp.dot(
            xb, w[DIM:, :], preferred_element_type=jnp.float32
        )
        v_ref[:, DIM:PAD] = jnp.zeros((blk, PAD - DIM), jnp.float32)

    return pl.pallas_call(
        body,
        grid=grid,
        in_specs=[
            pl.BlockSpec((blk, DIM), lambda i: (i, 0)),
            pl.BlockSpec((2 * DIM, DIM), lambda i: (0, 0)),
            pl.BlockSpec((1, DIM), lambda i: (0, 0)),
        ],
        out_specs=[
            pl.BlockSpec((blk, PAD), lambda i: (i, 0)),
            pl.BlockSpec((blk, PAD), lambda i: (i, 0)),
        ],
        out_shape=[
            jax.ShapeDtypeStruct((n_nodes, PAD), jnp.float32),
            jax.ShapeDtypeStruct((n_nodes, PAD), jnp.float32),
        ],
    )(x, W1, b1row)


def _make_sc_kernel(n_edges):
    per_w = n_edges // NW
    chunk = CHUNK
    n_chunks = per_w // chunk
    groups = chunk // DIM
    info = plsc.get_sparse_core_info()
    nc = info.num_cores
    mesh = plsc.VectorSubcoreMesh(core_axis_name="c", subcore_axis_name="s")

    @functools.partial(
        pl.kernel,
        mesh=mesh,
        out_type=jax.ShapeDtypeStruct((n_edges,), jnp.float32),
        scratch_types=[
            pltpu.VMEM((chunk,), jnp.int32),
            pltpu.VMEM((chunk,), jnp.int32),
            pltpu.VMEM((chunk, PAD), jnp.float32),
            pltpu.VMEM((chunk, PAD), jnp.float32),
            pltpu.VMEM((chunk,), jnp.float32),
            pltpu.VMEM((DIM, DIM), jnp.float32),
            pltpu.VMEM((DIM,), jnp.float32),
            pltpu.SemaphoreType.DMA,
            pltpu.SemaphoreType.DMA,
        ],
        compiler_params=pltpu.CompilerParams(
            needs_layout_passes=False, use_tc_tiling_on_sc=False
        ),
    )
    def sc_kernel(u_hbm, v_hbm, eh_hbm, et_hbm, w2s_hbm, b2s_hbm, out_hbm,
                  idxh_v, idxt_v, ubuf, vbuf, outbuf, w2v, b2v, sem_u, sem_v):
        wid = lax.axis_index("s") * nc + lax.axis_index("c")
        base0 = wid * per_w
        pltpu.sync_copy(w2s_hbm, w2v)
        pltpu.sync_copy(b2s_hbm, b2v)
        w2rows = [w2v[k] for k in range(DIM)]
        b2vec = b2v[...]
        iota16 = lax.iota(jnp.int32, DIM)

        def chunk_body(ci, carry):
            base = base0 + ci * chunk
            pltpu.sync_copy(eh_hbm.at[pl.ds(base, chunk)], idxh_v)
            pltpu.sync_copy(et_hbm.at[pl.ds(base, chunk)], idxt_v)
            cp_u = pltpu.async_copy(u_hbm.at[idxh_v], ubuf, sem_u)
            cp_v = pltpu.async_copy(v_hbm.at[idxt_v], vbuf, sem_v)
            cp_u.wait()
            cp_v.wait()

            def grp_body(g, c2):
                rows = g * DIM + iota16
                acc = b2vec
                for k in range(DIM):
                    colk = jnp.full((DIM,), k, jnp.int32)
                    uu = plsc.load_gather(ubuf, [rows, colk])
                    vv = plsc.load_gather(vbuf, [rows, colk])
                    acc = acc + jnp.maximum(uu + vv, 0.0) * w2rows[k]
                outbuf[pl.ds(g * DIM, DIM)] = 1.0 / (1.0 + jnp.exp(-acc))
                return c2

            lax.fori_loop(0, groups, grp_body, 0)
            pltpu.sync_copy(outbuf, out_hbm.at[pl.ds(base, chunk)])
            return carry

        lax.fori_loop(0, n_chunks, chunk_body, 0)

    return sc_kernel


def kernel(x, edge_label_index, W1, b1, W2, b2):
    n_edges = edge_label_index.shape[1]
    eh = edge_label_index[0].astype(jnp.int32)
    et = edge_label_index[1].astype(jnp.int32)
    U, V = _tc_precompute(x, W1, b1.reshape(1, DIM))
    w2s = jnp.broadcast_to(W2.reshape(DIM, 1), (DIM, DIM))
    b2s = jnp.broadcast_to(b2.reshape(1), (DIM,))
    out = _make_sc_kernel(n_edges)(U, V, eh, et, w2s, b2s)
    return out.reshape(n_edges, 1)


# speedup vs baseline: 38.0618x; 1.2832x over previous
import functools

import jax
import jax.numpy as jnp
from jax import lax
from jax.experimental import pallas as pl
from jax.experimental.pallas import tpu as pltpu
from jax.experimental.pallas import tpu_sc as plsc

DIM = 16
NW = 32
CHUNK = 2000


def _tc_precompute(x, W1, b1row):
    n_nodes = x.shape[0]
    blk = 5000
    grid = (n_nodes // blk,)

    def body(x_ref, w_ref, b_ref, u_ref, v_ref):
        xb = x_ref[...]
        w = w_ref[...]
        u_ref[...] = (
            jnp.dot(xb, w[:DIM, :], preferred_element_type=jnp.float32)
            + b_ref[...]
        )
        v_ref[...] = jnp.dot(xb, w[DIM:, :], preferred_element_type=jnp.float32)

    return pl.pallas_call(
        body,
        grid=grid,
        in_specs=[
            pl.BlockSpec((blk, DIM), lambda i: (i, 0)),
            pl.BlockSpec((2 * DIM, DIM), lambda i: (0, 0)),
            pl.BlockSpec((1, DIM), lambda i: (0, 0)),
        ],
        out_specs=[
            pl.BlockSpec((blk, DIM), lambda i: (i, 0)),
            pl.BlockSpec((blk, DIM), lambda i: (i, 0)),
        ],
        out_shape=[
            jax.ShapeDtypeStruct((n_nodes, DIM), jnp.float32),
            jax.ShapeDtypeStruct((n_nodes, DIM), jnp.float32),
        ],
    )(x, W1, b1row)


def _make_sc_kernel(n_edges):
    per_w = n_edges // NW
    chunk = CHUNK
    n_chunks = per_w // chunk
    groups = chunk // DIM
    nc = 2
    ns = 16
    mesh = plsc.VectorSubcoreMesh(
        core_axis_name="c", subcore_axis_name="s", num_cores=nc, num_subcores=ns
    )

    @functools.partial(
        pl.kernel,
        mesh=mesh,
        out_type=jax.ShapeDtypeStruct((n_edges,), jnp.float32),
        scratch_types=[
            pltpu.VMEM((chunk,), jnp.int32),
            pltpu.VMEM((chunk,), jnp.int32),
            pltpu.VMEM((chunk, DIM), jnp.float32),
            pltpu.VMEM((chunk, DIM), jnp.float32),
            pltpu.VMEM((chunk,), jnp.float32),
            pltpu.VMEM((DIM, DIM), jnp.float32),
            pltpu.VMEM((DIM,), jnp.float32),
            pltpu.SemaphoreType.DMA,
            pltpu.SemaphoreType.DMA,
        ],
        compiler_params=pltpu.CompilerParams(
            needs_layout_passes=False, use_tc_tiling_on_sc=False
        ),
    )
    def sc_kernel(u_hbm, v_hbm, eh_hbm, et_hbm, w2s_hbm, b2s_hbm, out_hbm,
                  idxh_v, idxt_v, ubuf, vbuf, outbuf, w2v, b2v, sem_u, sem_v):
        wid = lax.axis_index("s") * nc + lax.axis_index("c")
        base0 = wid * per_w
        pltpu.sync_copy(w2s_hbm, w2v)
        pltpu.sync_copy(b2s_hbm, b2v)
        w2diags = [w2v[d] for d in range(DIM)]
        b2vec = b2v[...]
        iota16 = lax.iota(jnp.int32, DIM)
        colvs = [jnp.bitwise_and(iota16 + d, DIM - 1) for d in range(DIM)]

        def chunk_body(ci, carry):
            base = base0 + ci * chunk
            pltpu.sync_copy(eh_hbm.at[pl.ds(base, chunk)], idxh_v)
            pltpu.sync_copy(et_hbm.at[pl.ds(base, chunk)], idxt_v)
            cp_u = pltpu.async_copy(u_hbm.at[idxh_v], ubuf, sem_u)
            cp_v = pltpu.async_copy(v_hbm.at[idxt_v], vbuf, sem_v)
            cp_u.wait()
            cp_v.wait()

            def grp_body(g, c2):
                rows = g * DIM + iota16
                acc = b2vec
                for d in range(DIM):
                    uu = plsc.load_gather(ubuf, [rows, colvs[d]])
                    vv = plsc.load_gather(vbuf, [rows, colvs[d]])
                    acc = acc + jnp.maximum(uu + vv, 0.0) * w2diags[d]
                outbuf[pl.ds(g * DIM, DIM)] = 1.0 / (1.0 + jnp.exp(-acc))
                return c2

            lax.fori_loop(0, groups, grp_body, 0)
            pltpu.sync_copy(outbuf, out_hbm.at[pl.ds(base, chunk)])
            return carry

        lax.fori_loop(0, n_chunks, chunk_body, 0)

    return sc_kernel


def kernel(x, edge_label_index, W1, b1, W2, b2):
    n_edges = edge_label_index.shape[1]
    eh = edge_label_index[0].astype(jnp.int32)
    et = edge_label_index[1].astype(jnp.int32)
    U, V = _tc_precompute(x, W1, b1.reshape(1, DIM))
    j = jnp.arange(DIM)
    w2s = W2.reshape(DIM)[(j[None, :] + j[:, None]) % DIM]
    b2s = jnp.broadcast_to(b2.reshape(1), (DIM,))
    out = _make_sc_kernel(n_edges)(U, V, eh, et, w2s, b2s)
    return out.reshape(n_edges, 1)


# speedup vs baseline: 45.4651x; 1.1945x over previous
import functools

import jax
import jax.numpy as jnp
from jax import lax
from jax.experimental import pallas as pl
from jax.experimental.pallas import tpu as pltpu
from jax.experimental.pallas import tpu_sc as plsc

DIM = 16
NW = 32
CHUNK = 400


def _tc_precompute(x, W1, b1row):
    n_nodes = x.shape[0]
    blk = 5000
    grid = (n_nodes // blk,)

    def body(x_ref, w_ref, b_ref, u_ref, v_ref):
        xb = x_ref[...]
        w = w_ref[...]
        u_ref[...] = (
            jnp.dot(xb, w[:DIM, :], preferred_element_type=jnp.float32)
            + b_ref[...]
        )
        v_ref[...] = jnp.dot(xb, w[DIM:, :], preferred_element_type=jnp.float32)

    return pl.pallas_call(
        body,
        grid=grid,
        in_specs=[
            pl.BlockSpec((blk, DIM), lambda i: (i, 0)),
            pl.BlockSpec((2 * DIM, DIM), lambda i: (0, 0)),
            pl.BlockSpec((1, DIM), lambda i: (0, 0)),
        ],
        out_specs=[
            pl.BlockSpec((blk, DIM), lambda i: (i, 0)),
            pl.BlockSpec((blk, DIM), lambda i: (i, 0)),
        ],
        out_shape=[
            jax.ShapeDtypeStruct((n_nodes, DIM), jnp.float32),
            jax.ShapeDtypeStruct((n_nodes, DIM), jnp.float32),
        ],
    )(x, W1, b1row)


def _make_sc_kernel(n_edges):
    per_w = n_edges // NW
    chunk = CHUNK
    n_chunks = per_w // chunk
    n_pairs = n_chunks // 2
    groups = chunk // DIM
    nc = 2
    ns = 16
    mesh = plsc.VectorSubcoreMesh(
        core_axis_name="c", subcore_axis_name="s", num_cores=nc, num_subcores=ns
    )

    @functools.partial(
        pl.kernel,
        mesh=mesh,
        out_type=jax.ShapeDtypeStruct((n_edges,), jnp.float32),
        scratch_types=[
            pltpu.VMEM((2, chunk), jnp.int32),
            pltpu.VMEM((2, chunk), jnp.int32),
            pltpu.VMEM((2 * chunk, DIM), jnp.float32),
            pltpu.VMEM((2 * chunk, DIM), jnp.float32),
            pltpu.VMEM((2, chunk), jnp.float32),
            pltpu.VMEM((DIM, DIM), jnp.float32),
            pltpu.VMEM((DIM,), jnp.float32),
            pltpu.SemaphoreType.DMA,
            pltpu.SemaphoreType.DMA,
            pltpu.SemaphoreType.DMA,
            pltpu.SemaphoreType.DMA,
            pltpu.SemaphoreType.DMA,
            pltpu.SemaphoreType.DMA,
        ],
        compiler_params=pltpu.CompilerParams(
            needs_layout_passes=False, use_tc_tiling_on_sc=False
        ),
    )
    def sc_kernel(u_hbm, v_hbm, eh_hbm, et_hbm, w2s_hbm, b2s_hbm, out_hbm,
                  idxh_v, idxt_v, ubuf, vbuf, outbuf, w2v, b2v,
                  si0, si1, sg0, sg1, so0, so1):
        wid = lax.axis_index("s") * nc + lax.axis_index("c")
        base0 = wid * per_w
        si = (si0, si1)
        sg = (sg0, sg1)
        so = (so0, so1)
        pltpu.sync_copy(w2s_hbm, w2v)
        pltpu.sync_copy(b2s_hbm, b2v)
        w2diags = [w2v[d] for d in range(DIM)]
        b2vec = b2v[...]
        iota16 = lax.iota(jnp.int32, DIM)
        colvs = [jnp.bitwise_and(iota16 + d, DIM - 1) for d in range(DIM)]

        def fire_idx(ci, b):
            base = base0 + ci * chunk
            pltpu.async_copy(eh_hbm.at[pl.ds(base, chunk)], idxh_v.at[b], si[b])
            pltpu.async_copy(et_hbm.at[pl.ds(base, chunk)], idxt_v.at[b], si[b])

        def fire_gather(b):
            pltpu.make_async_copy(
                eh_hbm.at[pl.ds(0, chunk)], idxh_v.at[b], si[b]).wait()
            pltpu.make_async_copy(
                et_hbm.at[pl.ds(0, chunk)], idxt_v.at[b], si[b]).wait()
            pltpu.async_copy(
                u_hbm.at[idxh_v.at[b]], ubuf.at[pl.ds(b * chunk, chunk)], sg[b])
            pltpu.async_copy(
                v_hbm.at[idxt_v.at[b]], vbuf.at[pl.ds(b * chunk, chunk)], sg[b])

        def drain_gather(b):
            pltpu.make_async_copy(
                u_hbm.at[idxh_v.at[b]],
                ubuf.at[pl.ds(b * chunk, chunk)], sg[b]).wait()
            pltpu.make_async_copy(
                v_hbm.at[idxt_v.at[b]],
                vbuf.at[pl.ds(b * chunk, chunk)], sg[b]).wait()

        def drain_out(b):
            pltpu.make_async_copy(
                outbuf.at[b], out_hbm.at[pl.ds(0, chunk)], so[b]).wait()

        def compute(ci, b):
            rbase = b * chunk

            def grp_body(g, c2):
                rows = rbase + g * DIM + iota16
                acc = b2vec
                for d in range(DIM):
                    uu = plsc.load_gather(ubuf, [rows, colvs[d]])
                    vv = plsc.load_gather(vbuf, [rows, colvs[d]])
                    acc = acc + jnp.maximum(uu + vv, 0.0) * w2diags[d]
                outbuf[b, pl.ds(g * DIM, DIM)] = 1.0 / (1.0 + jnp.exp(-acc))
                return c2

            lax.fori_loop(0, groups, grp_body, 0)
            pltpu.async_copy(
                outbuf.at[b],
                out_hbm.at[pl.ds(base0 + ci * chunk, chunk)], so[b])

        def phase(ci, b, p):
            drain_gather(b)
            pl.when(ci + 2 < n_chunks)(lambda: fire_idx(ci + 2, b))
            pl.when(p > 0)(lambda: drain_out(b))
            pl.when(ci + 1 < n_chunks)(lambda: fire_gather(1 - b))
            compute(ci, b)

        fire_idx(0, 0)
        fire_idx(1, 1)
        fire_gather(0)

        def pair_body(p, carry):
            phase(2 * p, 0, p)
            phase(2 * p + 1, 1, p)
            return carry

        lax.fori_loop(0, n_pairs, pair_body, 0)
        drain_out(0)
        drain_out(1)

    return sc_kernel


def kernel(x, edge_label_index, W1, b1, W2, b2):
    n_edges = edge_label_index.shape[1]
    eh = edge_label_index[0].astype(jnp.int32)
    et = edge_label_index[1].astype(jnp.int32)
    U, V = _tc_precompute(x, W1, b1.reshape(1, DIM))
    j = jnp.arange(DIM)
    w2s = W2.reshape(DIM)[(j[None, :] + j[:, None]) % DIM]
    b2s = jnp.broadcast_to(b2.reshape(1), (DIM,))
    out = _make_sc_kernel(n_edges)(U, V, eh, et, w2s, b2s)
    return out.reshape(n_edges, 1)


# speedup vs baseline: 55.4126x; 1.2188x over previous
import functools

import jax
import jax.numpy as jnp
from jax import lax
from jax.experimental import pallas as pl
from jax.experimental.pallas import tpu as pltpu
from jax.experimental.pallas import tpu_sc as plsc

DIM = 16
NW = 32
CHUNK = 400


def _tc_precompute(x2d, ba, bb, b1t):
    n_rows = x2d.shape[0]
    blk = n_rows
    grid = (1,)

    def body(x_ref, ba_ref, bb_ref, b_ref, u_ref, v_ref):
        xb = x_ref[...]
        u_ref[...] = (
            jnp.dot(xb, ba_ref[...], preferred_element_type=jnp.float32)
            + b_ref[...]
        )
        v_ref[...] = jnp.dot(xb, bb_ref[...], preferred_element_type=jnp.float32)

    return pl.pallas_call(
        body,
        grid=grid,
        in_specs=[
            pl.BlockSpec((blk, 8 * DIM), lambda i: (i, 0)),
            pl.BlockSpec((8 * DIM, 8 * DIM), lambda i: (0, 0)),
            pl.BlockSpec((8 * DIM, 8 * DIM), lambda i: (0, 0)),
            pl.BlockSpec((1, 8 * DIM), lambda i: (0, 0)),
        ],
        out_specs=[
            pl.BlockSpec((blk, 8 * DIM), lambda i: (i, 0)),
            pl.BlockSpec((blk, 8 * DIM), lambda i: (i, 0)),
        ],
        out_shape=[
            jax.ShapeDtypeStruct((n_rows, 8 * DIM), jnp.float32),
            jax.ShapeDtypeStruct((n_rows, 8 * DIM), jnp.float32),
        ],
    )(x2d, ba, bb, b1t)


def _make_sc_kernel(n_edges):
    per_w = n_edges // NW
    chunk = CHUNK
    n_chunks = per_w // chunk
    n_pairs = n_chunks // 2
    groups = chunk // DIM
    nc = 2
    ns = 16
    mesh = plsc.VectorSubcoreMesh(
        core_axis_name="c", subcore_axis_name="s", num_cores=nc, num_subcores=ns
    )

    @functools.partial(
        pl.kernel,
        mesh=mesh,
        out_type=jax.ShapeDtypeStruct((n_edges,), jnp.float32),
        scratch_types=[
            pltpu.VMEM((2, chunk), jnp.int32),
            pltpu.VMEM((2, chunk), jnp.int32),
            pltpu.VMEM((2 * chunk, DIM), jnp.float32),
            pltpu.VMEM((2 * chunk, DIM), jnp.float32),
            pltpu.VMEM((2, chunk), jnp.float32),
            pltpu.VMEM((DIM, DIM), jnp.float32),
            pltpu.VMEM((DIM,), jnp.float32),
            pltpu.SemaphoreType.DMA,
            pltpu.SemaphoreType.DMA,
            pltpu.SemaphoreType.DMA,
            pltpu.SemaphoreType.DMA,
            pltpu.SemaphoreType.DMA,
            pltpu.SemaphoreType.DMA,
        ],
        compiler_params=pltpu.CompilerParams(
            needs_layout_passes=False, use_tc_tiling_on_sc=False
        ),
    )
    def sc_kernel(u_hbm, v_hbm, el_hbm, w2s_hbm, b2s_hbm, out_hbm,
                  idxh_v, idxt_v, ubuf, vbuf, outbuf, w2v, b2v,
                  si0, si1, sg0, sg1, so0, so1):
        wid = lax.axis_index("s") * nc + lax.axis_index("c")
        base0 = wid * per_w
        si = (si0, si1)
        sg = (sg0, sg1)
        so = (so0, so1)
        pltpu.sync_copy(w2s_hbm, w2v)
        pltpu.sync_copy(b2s_hbm, b2v)
        w2diags = [w2v[d] for d in range(DIM)]
        b2vec = b2v[...]
        iota16 = lax.iota(jnp.int32, DIM)
        colvs = [jnp.bitwise_and(iota16 + d, DIM - 1) for d in range(DIM)]

        def fire_idx(ci, b):
            base = base0 + ci * chunk
            pltpu.async_copy(el_hbm.at[0, pl.ds(base, chunk)], idxh_v.at[b], si[b])
            pltpu.async_copy(el_hbm.at[1, pl.ds(base, chunk)], idxt_v.at[b], si[b])

        def fire_gather(b):
            pltpu.make_async_copy(
                el_hbm.at[0, pl.ds(0, chunk)], idxh_v.at[b], si[b]).wait()
            pltpu.make_async_copy(
                el_hbm.at[1, pl.ds(0, chunk)], idxt_v.at[b], si[b]).wait()
            pltpu.async_copy(
                u_hbm.at[idxh_v.at[b]], ubuf.at[pl.ds(b * chunk, chunk)], sg[b])
            pltpu.async_copy(
                v_hbm.at[idxt_v.at[b]], vbuf.at[pl.ds(b * chunk, chunk)], sg[b])

        def drain_gather(b):
            pltpu.make_async_copy(
                u_hbm.at[idxh_v.at[b]],
                ubuf.at[pl.ds(b * chunk, chunk)], sg[b]).wait()
            pltpu.make_async_copy(
                v_hbm.at[idxt_v.at[b]],
                vbuf.at[pl.ds(b * chunk, chunk)], sg[b]).wait()

        def drain_out(b):
            pltpu.make_async_copy(
                outbuf.at[b], out_hbm.at[pl.ds(0, chunk)], so[b]).wait()

        def compute(ci, b):
            rbase = b * chunk

            def grp_body(g, c2):
                rows = rbase + g * DIM + iota16
                acc = b2vec
                for d in range(DIM):
                    uu = plsc.load_gather(ubuf, [rows, colvs[d]])
                    vv = plsc.load_gather(vbuf, [rows, colvs[d]])
                    acc = acc + jnp.maximum(uu + vv, 0.0) * w2diags[d]
                outbuf[b, pl.ds(g * DIM, DIM)] = 1.0 / (1.0 + jnp.exp(-acc))
                return c2

            lax.fori_loop(0, groups, grp_body, 0)
            pltpu.async_copy(
                outbuf.at[b],
                out_hbm.at[pl.ds(base0 + ci * chunk, chunk)], so[b])

        def phase(ci, b, p):
            drain_gather(b)
            pl.when(ci + 2 < n_chunks)(lambda: fire_idx(ci + 2, b))
            pl.when(p > 0)(lambda: drain_out(b))
            pl.when(ci + 1 < n_chunks)(lambda: fire_gather(1 - b))
            compute(ci, b)

        fire_idx(0, 0)
        fire_idx(1, 1)
        fire_gather(0)

        def pair_body(p, carry):
            phase(2 * p, 0, p)
            phase(2 * p + 1, 1, p)
            return carry

        lax.fori_loop(0, n_pairs, pair_body, 0)
        drain_out(0)
        drain_out(1)

    return sc_kernel


def kernel(x, edge_label_index, W1, b1, W2, b2):
    n_nodes = x.shape[0]
    n_edges = edge_label_index.shape[1]
    el = edge_label_index.astype(jnp.int32)
    x2d = x.reshape(n_nodes // 8, 8 * DIM)
    eye8 = jnp.eye(8, dtype=jnp.float32)
    ba = jnp.kron(eye8, W1[:DIM, :])
    bb = jnp.kron(eye8, W1[DIM:, :])
    b1t = jnp.tile(b1, 8).reshape(1, 8 * DIM)
    U2d, V2d = _tc_precompute(x2d, ba, bb, b1t)
    U = U2d.reshape(n_nodes, DIM)
    V = V2d.reshape(n_nodes, DIM)
    j = jnp.arange(DIM)
    w2s = W2.reshape(DIM)[(j[None, :] + j[:, None]) % DIM]
    b2s = jnp.broadcast_to(b2.reshape(1), (DIM,))
    out = _make_sc_kernel(n_edges)(U, V, el, w2s, b2s)
    return out.reshape(n_edges, 1)


# speedup vs baseline: 66.8724x; 1.2068x over previous
import functools

import jax
import jax.numpy as jnp
from jax import lax
from jax.experimental import pallas as pl
from jax.experimental.pallas import tpu as pltpu
from jax.experimental.pallas import tpu_sc as plsc

DIM = 16
NW = 32
CHUNK = 800


def _tc_precompute(x2d, ba, bb, b1t):
    n_rows = x2d.shape[0]
    blk = n_rows
    grid = (1,)

    def body(x_ref, ba_ref, bb_ref, b_ref, u_ref, v_ref):
        xb = x_ref[...]
        u_ref[...] = (
            jnp.dot(xb, ba_ref[...], preferred_element_type=jnp.float32)
            + b_ref[...]
        )
        v_ref[...] = jnp.dot(xb, bb_ref[...], preferred_element_type=jnp.float32)

    return pl.pallas_call(
        body,
        grid=grid,
        in_specs=[
            pl.BlockSpec((blk, 8 * DIM), lambda i: (i, 0)),
            pl.BlockSpec((8 * DIM, 8 * DIM), lambda i: (0, 0)),
            pl.BlockSpec((8 * DIM, 8 * DIM), lambda i: (0, 0)),
            pl.BlockSpec((1, 8 * DIM), lambda i: (0, 0)),
        ],
        out_specs=[
            pl.BlockSpec((blk, 8 * DIM), lambda i: (i, 0)),
            pl.BlockSpec((blk, 8 * DIM), lambda i: (i, 0)),
        ],
        out_shape=[
            jax.ShapeDtypeStruct((n_rows, 8 * DIM), jnp.float32),
            jax.ShapeDtypeStruct((n_rows, 8 * DIM), jnp.float32),
        ],
    )(x2d, ba, bb, b1t)


def _make_sc_kernel(n_edges):
    per_w = n_edges // NW
    chunk = CHUNK
    n_chunks = per_w // chunk
    n_pairs = n_chunks // 2
    groups = chunk // DIM
    nc = 2
    ns = 16
    mesh = plsc.VectorSubcoreMesh(
        core_axis_name="c", subcore_axis_name="s", num_cores=nc, num_subcores=ns
    )

    @functools.partial(
        pl.kernel,
        mesh=mesh,
        out_type=jax.ShapeDtypeStruct((n_edges,), jnp.float32),
        scratch_types=[
            pltpu.VMEM((2, chunk), jnp.int32),
            pltpu.VMEM((2, chunk), jnp.int32),
            pltpu.VMEM((2 * chunk, DIM), jnp.float32),
            pltpu.VMEM((2 * chunk, DIM), jnp.float32),
            pltpu.VMEM((2, chunk), jnp.float32),
            pltpu.VMEM((DIM, DIM), jnp.float32),
            pltpu.VMEM((DIM,), jnp.float32),
            pltpu.SemaphoreType.DMA,
            pltpu.SemaphoreType.DMA,
            pltpu.SemaphoreType.DMA,
            pltpu.SemaphoreType.DMA,
            pltpu.SemaphoreType.DMA,
            pltpu.SemaphoreType.DMA,
        ],
        compiler_params=pltpu.CompilerParams(
            needs_layout_passes=False, use_tc_tiling_on_sc=False
        ),
    )
    def sc_kernel(u_hbm, v_hbm, el_hbm, w2s_hbm, b2s_hbm, out_hbm,
                  idxh_v, idxt_v, ubuf, vbuf, outbuf, w2v, b2v,
                  si0, si1, sg0, sg1, so0, so1):
        wid = lax.axis_index("s") * nc + lax.axis_index("c")
        base0 = wid * per_w
        si = (si0, si1)
        sg = (sg0, sg1)
        so = (so0, so1)
        pltpu.sync_copy(w2s_hbm, w2v)
        pltpu.sync_copy(b2s_hbm, b2v)
        w2diags = [w2v[d] for d in range(DIM)]
        b2vec = b2v[...]
        iota16 = lax.iota(jnp.int32, DIM)
        colvs = [jnp.bitwise_and(iota16 + d, DIM - 1) for d in range(DIM)]

        def fire_idx(ci, b):
            base = base0 + ci * chunk
            pltpu.async_copy(el_hbm.at[0, pl.ds(base, chunk)], idxh_v.at[b], si[b])
            pltpu.async_copy(el_hbm.at[1, pl.ds(base, chunk)], idxt_v.at[b], si[b])

        def fire_gather(b):
            pltpu.make_async_copy(
                el_hbm.at[0, pl.ds(0, chunk)], idxh_v.at[b], si[b]).wait()
            pltpu.make_async_copy(
                el_hbm.at[1, pl.ds(0, chunk)], idxt_v.at[b], si[b]).wait()
            pltpu.async_copy(
                u_hbm.at[idxh_v.at[b]], ubuf.at[pl.ds(b * chunk, chunk)], sg[b])
            pltpu.async_copy(
                v_hbm.at[idxt_v.at[b]], vbuf.at[pl.ds(b * chunk, chunk)], sg[b])

        def drain_gather(b):
            pltpu.make_async_copy(
                u_hbm.at[idxh_v.at[b]],
                ubuf.at[pl.ds(b * chunk, chunk)], sg[b]).wait()
            pltpu.make_async_copy(
                v_hbm.at[idxt_v.at[b]],
                vbuf.at[pl.ds(b * chunk, chunk)], sg[b]).wait()

        def drain_out(b):
            pltpu.make_async_copy(
                outbuf.at[b], out_hbm.at[pl.ds(0, chunk)], so[b]).wait()

        def compute(ci, b):
            rbase = b * chunk

            def grp_body(g, c2):
                rows = rbase + g * DIM + iota16
                acc = b2vec
                for d in range(DIM):
                    uu = plsc.load_gather(ubuf, [rows, colvs[d]])
                    vv = plsc.load_gather(vbuf, [rows, colvs[d]])
                    acc = acc + jnp.maximum(uu + vv, 0.0) * w2diags[d]
                outbuf[b, pl.ds(g * DIM, DIM)] = 1.0 / (1.0 + jnp.exp(-acc))
                return c2

            lax.fori_loop(0, groups, grp_body, 0)
            pltpu.async_copy(
                outbuf.at[b],
                out_hbm.at[pl.ds(base0 + ci * chunk, chunk)], so[b])

        def phase(ci, b, p):
            drain_gather(b)
            pl.when(ci + 2 < n_chunks)(lambda: fire_idx(ci + 2, b))
            pl.when(p > 0)(lambda: drain_out(b))
            pl.when(ci + 1 < n_chunks)(lambda: fire_gather(1 - b))
            compute(ci, b)

        fire_idx(0, 0)
        fire_idx(1, 1)
        fire_gather(0)

        def pair_body(p, carry):
            phase(2 * p, 0, p)
            phase(2 * p + 1, 1, p)
            return carry

        lax.fori_loop(0, n_pairs, pair_body, 0)
        if n_chunks % 2:
            phase(n_chunks - 1, 0, n_pairs)
        drain_out(0)
        drain_out(1)

    return sc_kernel


def kernel(x, edge_label_index, W1, b1, W2, b2):
    n_nodes = x.shape[0]
    n_edges = edge_label_index.shape[1]
    el = edge_label_index.astype(jnp.int32)
    x2d = x.reshape(n_nodes // 8, 8 * DIM)
    eye8 = jnp.eye(8, dtype=jnp.float32)
    ba = jnp.kron(eye8, W1[:DIM, :])
    bb = jnp.kron(eye8, W1[DIM:, :])
    b1t = jnp.tile(b1, 8).reshape(1, 8 * DIM)
    U2d, V2d = _tc_precompute(x2d, ba, bb, b1t)
    U = U2d.reshape(n_nodes, DIM)
    V = V2d.reshape(n_nodes, DIM)
    j = jnp.arange(DIM)
    w2s = W2.reshape(DIM)[(j[None, :] + j[:, None]) % DIM]
    b2s = jnp.broadcast_to(b2.reshape(1), (DIM,))
    out = _make_sc_kernel(n_edges)(U, V, el, w2s, b2s)
    return out.reshape(n_edges, 1)


# speedup vs baseline: 78.6205x; 1.1757x over previous
import functools

import jax
import jax.numpy as jnp
from jax import lax
from jax.experimental import pallas as pl
from jax.experimental.pallas import tpu as pltpu
from jax.experimental.pallas import tpu_sc as plsc

DIM = 16
NW = 32
CHUNK = 800


def _tc_precompute(x2d, ba, bb, b1t):
    n_rows = x2d.shape[0]
    blk = n_rows
    grid = (1,)

    def body(x_ref, ba_ref, bb_ref, b_ref, u_ref, v_ref):
        xb = x_ref[...]
        u_ref[...] = (
            jnp.dot(xb, ba_ref[...], preferred_element_type=jnp.float32)
            + b_ref[...]
        )
        v_ref[...] = jnp.dot(xb, bb_ref[...], preferred_element_type=jnp.float32)

    return pl.pallas_call(
        body,
        grid=grid,
        in_specs=[
            pl.BlockSpec((blk, 8 * DIM), lambda i: (i, 0)),
            pl.BlockSpec((8 * DIM, 8 * DIM), lambda i: (0, 0)),
            pl.BlockSpec((8 * DIM, 8 * DIM), lambda i: (0, 0)),
            pl.BlockSpec((1, 8 * DIM), lambda i: (0, 0)),
        ],
        out_specs=[
            pl.BlockSpec((blk, 8 * DIM), lambda i: (i, 0)),
            pl.BlockSpec((blk, 8 * DIM), lambda i: (i, 0)),
        ],
        out_shape=[
            jax.ShapeDtypeStruct((n_rows, 8 * DIM), jnp.float32),
            jax.ShapeDtypeStruct((n_rows, 8 * DIM), jnp.float32),
        ],
    )(x2d, ba, bb, b1t)


def _make_sc_kernel(n_edges):
    per_w = n_edges // NW
    chunk = CHUNK
    n_chunks = per_w // chunk
    n_tris = n_chunks // 3
    groups = chunk // DIM
    nc = 2
    ns = 16
    mesh = plsc.VectorSubcoreMesh(
        core_axis_name="c", subcore_axis_name="s", num_cores=nc, num_subcores=ns
    )

    @functools.partial(
        pl.kernel,
        mesh=mesh,
        out_type=jax.ShapeDtypeStruct((n_edges,), jnp.float32),
        scratch_types=[
            pltpu.VMEM((3, chunk), jnp.int32),
            pltpu.VMEM((3, chunk), jnp.int32),
            pltpu.VMEM((3 * chunk, DIM), jnp.float32),
            pltpu.VMEM((3 * chunk, DIM), jnp.float32),
            pltpu.VMEM((3, chunk), jnp.float32),
            pltpu.VMEM((DIM, DIM), jnp.float32),
            pltpu.VMEM((DIM,), jnp.float32),
            pltpu.SemaphoreType.DMA,
            pltpu.SemaphoreType.DMA,
            pltpu.SemaphoreType.DMA,
            pltpu.SemaphoreType.DMA,
            pltpu.SemaphoreType.DMA,
            pltpu.SemaphoreType.DMA,
            pltpu.SemaphoreType.DMA,
            pltpu.SemaphoreType.DMA,
            pltpu.SemaphoreType.DMA,
        ],
        compiler_params=pltpu.CompilerParams(
            needs_layout_passes=False, use_tc_tiling_on_sc=False
        ),
    )
    def sc_kernel(u_hbm, v_hbm, el_hbm, w2s_hbm, b2s_hbm, out_hbm,
                  idxh_v, idxt_v, ubuf, vbuf, outbuf, w2v, b2v,
                  si0, si1, si2, sg0, sg1, sg2, so0, so1, so2):
        wid = lax.axis_index("s") * nc + lax.axis_index("c")
        base0 = wid * per_w
        si = (si0, si1, si2)
        sg = (sg0, sg1, sg2)
        so = (so0, so1, so2)
        pltpu.sync_copy(w2s_hbm, w2v)
        pltpu.sync_copy(b2s_hbm, b2v)
        w2diags = [w2v[d] for d in range(DIM)]
        b2vec = b2v[...]
        iota16 = lax.iota(jnp.int32, DIM)
        colvs = [jnp.bitwise_and(iota16 + d, DIM - 1) for d in range(DIM)]

        def fire_idx(ci, b):
            base = base0 + ci * chunk
            pltpu.async_copy(el_hbm.at[0, pl.ds(base, chunk)], idxh_v.at[b], si[b])
            pltpu.async_copy(el_hbm.at[1, pl.ds(base, chunk)], idxt_v.at[b], si[b])

        def fire_gather(b):
            pltpu.make_async_copy(
                el_hbm.at[0, pl.ds(0, chunk)], idxh_v.at[b], si[b]).wait()
            pltpu.make_async_copy(
                el_hbm.at[1, pl.ds(0, chunk)], idxt_v.at[b], si[b]).wait()
            pltpu.async_copy(
                u_hbm.at[idxh_v.at[b]], ubuf.at[pl.ds(b * chunk, chunk)], sg[b])
            pltpu.async_copy(
                v_hbm.at[idxt_v.at[b]], vbuf.at[pl.ds(b * chunk, chunk)], sg[b])

        def drain_gather(b):
            pltpu.make_async_copy(
                u_hbm.at[idxh_v.at[b]],
                ubuf.at[pl.ds(b * chunk, chunk)], sg[b]).wait()
            pltpu.make_async_copy(
                v_hbm.at[idxt_v.at[b]],
                vbuf.at[pl.ds(b * chunk, chunk)], sg[b]).wait()

        def drain_out(b):
            pltpu.make_async_copy(
                outbuf.at[b], out_hbm.at[pl.ds(0, chunk)], so[b]).wait()

        def compute(ci, b):
            rbase = b * chunk

            def grp_body(g, c2):
                rows = rbase + g * DIM + iota16
                acc = b2vec
                for d in range(DIM):
                    uu = plsc.load_gather(ubuf, [rows, colvs[d]])
                    vv = plsc.load_gather(vbuf, [rows, colvs[d]])
                    acc = acc + jnp.maximum(uu + vv, 0.0) * w2diags[d]
                outbuf[b, pl.ds(g * DIM, DIM)] = 1.0 / (1.0 + jnp.exp(-acc))
                return c2

            lax.fori_loop(0, groups, grp_body, 0)
            pltpu.async_copy(
                outbuf.at[b],
                out_hbm.at[pl.ds(base0 + ci * chunk, chunk)], so[b])

        def maybe(cond, fn):
            if isinstance(cond, bool):
                if cond:
                    fn()
            else:
                pl.when(cond)(fn)

        def phase(ci, b):
            drain_gather(b)
            maybe(ci + 3 < n_chunks, lambda: fire_idx(ci + 3, b))
            maybe(ci >= 3, lambda: drain_out(b))
            maybe(ci + 2 < n_chunks,
                  lambda: fire_gather((b + 2) % 3))
            compute(ci, b)

        fire_idx(0, 0)
        fire_idx(1, 1)
        fire_idx(2, 2)
        fire_gather(0)
        fire_gather(1)

        def tri_body(t, carry):
            phase(3 * t, 0)
            phase(3 * t + 1, 1)
            phase(3 * t + 2, 2)
            return carry

        lax.fori_loop(0, n_tris, tri_body, 0)
        for ci in range(3 * n_tris, n_chunks):
            phase(ci, ci % 3)
        for tb in range(3):
            drain_out((n_chunks - 3 + tb) % 3)

    return sc_kernel


def kernel(x, edge_label_index, W1, b1, W2, b2):
    n_nodes = x.shape[0]
    n_edges = edge_label_index.shape[1]
    el = edge_label_index.astype(jnp.int32)
    x2d = x.reshape(n_nodes // 8, 8 * DIM)
    eye8 = jnp.eye(8, dtype=jnp.float32)
    ba = jnp.kron(eye8, W1[:DIM, :])
    bb = jnp.kron(eye8, W1[DIM:, :])
    b1t = jnp.tile(b1, 8).reshape(1, 8 * DIM)
    U2d, V2d = _tc_precompute(x2d, ba, bb, b1t)
    U = U2d.reshape(n_nodes, DIM)
    V = V2d.reshape(n_nodes, DIM)
    j = jnp.arange(DIM)
    w2s = W2.reshape(DIM)[(j[None, :] + j[:, None]) % DIM]
    b2s = jnp.broadcast_to(b2.reshape(1), (DIM,))
    out = _make_sc_kernel(n_edges)(U, V, el, w2s, b2s)
    return out.reshape(n_edges, 1)
